# Initial kernel scaffold; baseline (speedup 1.0000x reference)
import jax
import jax.numpy as jnp
from jax.experimental import pallas as pl


def kernel(feats, xyz, index_0, index_1, index_0_offsets, n_max, qkv_w, qkv_b, proj_w, proj_b, rel_q_table, rel_k_table):
    raise NotImplementedError("write your pallas kernel here")


# speedup vs baseline: 17.1933x; 17.1933x over previous
import functools

import jax
import jax.numpy as jnp
from jax import lax
from jax.experimental import pallas as pl
from jax.experimental.pallas import tpu as pltpu, tpu_sc as plsc

N = 10000
DIM = 128
NUM_HEADS = 8
HEAD_DIM = 16
M = 320000
SCALE_Q = float(HEAD_DIM) ** -0.5
QUANT = 0.075
WINDOW = 0.6
NQT = 16
TBL = 2 * NQT
NW = 32
QPW = 313
LMAX = 10768
P = 64
NG = P // 16
NPAD = NW * QPW


def _tc_prep(feats, qkv_wt, qkv_b, tq, tk):
    blk = 1000
    grid = N // blk

    def body(f_ref, w_ref, b_ref, tq_ref, tk_ref, q_ref, k_ref, v_ref,
             qd_ref, kd_ref):
        f = f_ref[...]
        z = jnp.dot(f, w_ref[...], preferred_element_type=jnp.float32)
        z = z + b_ref[...][None, :]
        q = z[:, :DIM] * SCALE_Q
        k = z[:, DIM:2 * DIM]
        q_ref[...] = q
        k_ref[...] = k
        v_ref[...] = z[:, 2 * DIM:]
        qd_ref[...] = jnp.dot(q, tq_ref[...], preferred_element_type=jnp.float32)
        kd_ref[...] = jnp.dot(k, tk_ref[...], preferred_element_type=jnp.float32)

    full = lambda s: pl.BlockSpec(s, lambda i: (0, 0))
    row = lambda w: pl.BlockSpec((blk, w), lambda i: (i, 0))
    return pl.pallas_call(
        body,
        grid=(grid,),
        in_specs=[row(DIM), full((DIM, 3 * DIM)),
                  pl.BlockSpec((3 * DIM,), lambda i: (0,)),
                  full((DIM, 768)), full((DIM, 768))],
        out_specs=[row(DIM), row(DIM), row(DIM), row(768), row(768)],
        out_shape=[
            jax.ShapeDtypeStruct((N, DIM), jnp.float32),
            jax.ShapeDtypeStruct((N, DIM), jnp.float32),
            jax.ShapeDtypeStruct((N, DIM), jnp.float32),
            jax.ShapeDtypeStruct((N, 768), jnp.float32),
            jax.ShapeDtypeStruct((N, 768), jnp.float32),
        ],
    )(feats, qkv_wt, qkv_b, tq, tk)


def _tc_proj(x, proj_wt, proj_b):
    blk = 1000

    def body(x_ref, w_ref, b_ref, o_ref):
        o_ref[...] = (
            jnp.dot(x_ref[...], w_ref[...], preferred_element_type=jnp.float32)
            + b_ref[...][None, :])

    return pl.pallas_call(
        body,
        grid=(N // blk,),
        in_specs=[pl.BlockSpec((blk, DIM), lambda i: (i, 0)),
                  pl.BlockSpec((DIM, DIM), lambda i: (0, 0)),
                  pl.BlockSpec((DIM,), lambda i: (0,))],
        out_specs=pl.BlockSpec((blk, DIM), lambda i: (i, 0)),
        out_shape=jax.ShapeDtypeStruct((N, DIM), jnp.float32),
    )(x, proj_wt, proj_b)


def _sc_main(offs_pad, idx0_pad, idx1_pad, q2, k2, v2, xyz8, qd2, kd2):
    mesh = plsc.VectorSubcoreMesh(core_axis_name="c", subcore_axis_name="s",
                                  num_cores=2, num_subcores=16)
    f32, i32 = jnp.float32, jnp.int32

    @functools.partial(
        pl.kernel,
        out_type=jax.ShapeDtypeStruct((NW * QPW * DIM,), f32),
        mesh=mesh,
        compiler_params=pltpu.CompilerParams(
            needs_layout_passes=False, use_tc_tiling_on_sc=False),
        scratch_types=[
            pltpu.VMEM((352,), i32),
            pltpu.VMEM((LMAX,), i32),
            pltpu.VMEM((LMAX,), i32),
            pltpu.VMEM((P, DIM), f32),
            pltpu.VMEM((P, DIM), f32),
            pltpu.VMEM((P, DIM), f32),
            pltpu.VMEM((P, 16), f32),
            pltpu.VMEM((P, 16), f32),
            [pltpu.VMEM((P,), i32) for _ in range(3)],
            [pltpu.VMEM((P,), i32) for _ in range(3)],
            [pltpu.VMEM((P, 16), f32) for _ in range(3)],
            [pltpu.VMEM((P, 16), f32) for _ in range(3)],
            pltpu.VMEM((QPW * DIM + 128,), f32),
            pltpu.VMEM((QPW * 16 + 16,), f32),
            pltpu.SemaphoreType.DMA,
            pltpu.SemaphoreType.DMA,
        ],
    )
    def body(offs_hbm, idx0_hbm, idx1_hbm, q2_hbm, k2_hbm, v2_hbm,
             xyz_hbm, qd_hbm, kd_hbm, out_hbm,
             offs_v, idx0_v, idx1_v, qb, kb, vb, xy0, xy1,
             qdi, kdi, qdb, kdb, xacc, sacc, sem0, sem1):
        LANE = jnp.arange(16, dtype=i32)
        cc = lax.axis_index("c")
        ss = lax.axis_index("s")
        wid = ss * 2 + cc
        qlo = wid * QPW
        qhi = jnp.minimum(qlo + QPW, N)
        base = (qlo // 8) * 8
        pltpu.sync_copy(offs_hbm.at[pl.ds(base, 352)], offs_v)
        plo = offs_v[pl.ds(qlo - base, 16)][0]
        phi = offs_v[pl.ds(qhi - base, 16)][0]
        pbase = (plo // 8) * 8
        off0 = plo - pbase
        jend = off0 + phi - plo
        pltpu.sync_copy(idx0_hbm.at[pl.ds(pbase, LMAX)], idx0_v)
        pltpu.sync_copy(idx1_hbm.at[pl.ds(pbase, LMAX)], idx1_v)

        zf = jnp.zeros((16,), f32)

        def zx(i, carry):
            xacc[pl.ds(i * 16, 16)] = zf
            return carry

        lax.fori_loop(0, (QPW * DIM + 128) // 16, zx, 0)

        def zs(i, carry):
            sacc[pl.ds(i * 16, 16)] = zf
            return carry

        lax.fori_loop(0, (QPW * 16 + 16) // 16, zs, 0)

        nchunks = (jend + P - 1) // P

        def chunk(ci, carry):
            cps = [
                pltpu.async_copy(q2_hbm.at[idx0_v.at[pl.ds(ci * P, P)]],
                                 qb, sem0),
                pltpu.async_copy(k2_hbm.at[idx1_v.at[pl.ds(ci * P, P)]],
                                 kb, sem0),
                pltpu.async_copy(v2_hbm.at[idx1_v.at[pl.ds(ci * P, P)]],
                                 vb, sem0),
                pltpu.async_copy(xyz_hbm.at[idx0_v.at[pl.ds(ci * P, P)]],
                                 xy0, sem0),
                pltpu.async_copy(xyz_hbm.at[idx1_v.at[pl.ds(ci * P, P)]],
                                 xy1, sem0),
            ]
            for cp in cps:
                cp.wait()
            for g in range(NG):
                j16 = g * 16 + LANE
                i0 = idx0_v[pl.ds(ci * P + g * 16, 16)]
                i1 = idx1_v[pl.ds(ci * P + g * 16, 16)]
                for c3 in range(3):
                    cvec = jnp.full((16,), c3, i32)
                    x0 = plsc.load_gather(xy0, [j16, cvec])
                    x1 = plsc.load_gather(xy1, [j16, cvec])
                    rel = x0 - x1
                    y = rel * 100000.0
                    r = (y + jnp.sign(y) * 0.5).astype(i32).astype(f32)
                    u = (r * 1e-5 + (2.0 * WINDOW - 0.0001)) * (1.0 / QUANT)
                    t = jnp.clip(u.astype(i32), 0, TBL - 1)
                    qdi[c3][pl.ds(g * 16, 16)] = i0 * 96 + c3 * TBL + t
                    kdi[c3][pl.ds(g * 16, 16)] = i1 * 96 + c3 * TBL + t
            cps = [pltpu.async_copy(qd_hbm.at[qdi[c3]], qdb[c3], sem1)
                   for c3 in range(3)]
            cps += [pltpu.async_copy(kd_hbm.at[kdi[c3]], kdb[c3], sem1)
                    for c3 in range(3)]
            for cp in cps:
                cp.wait()
            for g in range(NG):
                j16 = g * 16 + LANE
                i0 = idx0_v[pl.ds(ci * P + g * 16, 16)]
                jj = ci * P + g * 16 + LANE
                ok = (jj >= off0) & (jj < jend)
                seg = jnp.where(ok, i0 - qlo, 0)
                accs = [zf] * NUM_HEADS
                for d in range(DIM):
                    dv = jnp.full((16,), d, i32)
                    qv = plsc.load_gather(qb, [j16, dv])
                    kv = plsc.load_gather(kb, [j16, dv])
                    accs[d // HEAD_DIM] = accs[d // HEAD_DIM] + qv * kv
                es = []
                for h in range(NUM_HEADS):
                    hv = jnp.full((16,), h, i32)
                    bias = zf
                    for c3 in range(3):
                        bias = bias + plsc.load_gather(qdb[c3], [j16, hv])
                        bias = bias + plsc.load_gather(kdb[c3], [j16, hv])
                    e = jnp.exp(accs[h] + bias)
                    es.append(jnp.where(ok, e, 0.0))
                onehots = [jnp.where(LANE == h, 1.0, 0.0).astype(f32)
                           for h in range(NUM_HEADS)]
                for j in range(16):
                    segj = seg[j]
                    rowb = segj * DIM
                    svec = zf
                    for h in range(NUM_HEADS):
                        svec = svec + es[h][j] * onehots[h]
                    plsc.addupdate(sacc.at[pl.ds(segj * 16, 16)], svec)
                    for h in range(NUM_HEADS):
                        vrow = vb[g * 16 + j, pl.ds(h * 16, 16)]
                        plsc.addupdate(xacc.at[pl.ds(rowb + h * 16, 16)],
                                       es[h][j] * vrow)
            return carry

        lax.fori_loop(0, nchunks, chunk, 0)

        def fin(i, carry):
            s16 = sacc[pl.ds(i * 16, 16)]
            w16 = 1.0 / jnp.maximum(s16, 1e-30)
            for h in range(NUM_HEADS):
                sl = pl.ds(i * DIM + h * 16, 16)
                xacc[sl] = xacc[sl] * w16[h]
            return carry

        lax.fori_loop(0, QPW, fin, 0)
        pltpu.sync_copy(xacc.at[pl.ds(0, QPW * DIM)],
                        out_hbm.at[pl.ds(qlo * DIM, QPW * DIM)])

    return body(offs_pad, idx0_pad, idx1_pad, q2, k2, v2, xyz8, qd2, kd2)


def kernel(feats, xyz, index_0, index_1, index_0_offsets, n_max,
           qkv_w, qkv_b, proj_w, proj_b, rel_q_table, rel_k_table):
    f32, i32 = jnp.float32, jnp.int32
    eye = jnp.eye(NUM_HEADS, dtype=f32)
    tq = jnp.einsum("cthd,hg->hdctg",
                    jnp.transpose(rel_q_table, (3, 0, 1, 2)), eye)
    tq = tq.reshape(DIM, 3 * TBL * NUM_HEADS)
    tk = jnp.einsum("cthd,hg->hdctg",
                    jnp.transpose(rel_k_table, (3, 0, 1, 2)), eye)
    tk = tk.reshape(DIM, 3 * TBL * NUM_HEADS)

    q2, k2, v2, qd, kd = _tc_prep(feats, qkv_w.T, qkv_b, tq, tk)
    qd2 = jnp.pad(qd.reshape(N * 96, 8), ((0, 0), (0, 8)))
    kd2 = jnp.pad(kd.reshape(N * 96, 8), ((0, 0), (0, 8)))

    xyz8 = jnp.pad(xyz, ((0, 0), (0, 13)))
    offs_pad = jnp.pad(index_0_offsets.astype(i32), (0, 352))
    idx0_pad = jnp.pad(index_0.astype(i32), (0, LMAX + 8))
    idx1_pad = jnp.pad(index_1.astype(i32), (0, LMAX + 8))

    xraw = _sc_main(offs_pad, idx0_pad, idx1_pad, q2, k2, v2, xyz8, qd2, kd2)
    return _tc_proj(xraw.reshape(NPAD, DIM)[:N], proj_w.T, proj_b)


# speedup vs baseline: 17.7258x; 1.0310x over previous
import functools

import jax
import jax.numpy as jnp
from jax import lax
from jax.experimental import pallas as pl
from jax.experimental.pallas import tpu as pltpu, tpu_sc as plsc

N = 10000
DIM = 128
NUM_HEADS = 8
HEAD_DIM = 16
M = 320000
SCALE_Q = float(HEAD_DIM) ** -0.5
QUANT = 0.075
WINDOW = 0.6
NQT = 16
TBL = 2 * NQT
NW = 32
QPW = 313
LMAX = 10880
P = 48
NG = P // 16
NPAD = NW * QPW


def _tc_prep(feats, qkv_wt, qkv_b, tq, tk):
    blk = 1000
    grid = N // blk

    def body(f_ref, w_ref, b_ref, tq_ref, tk_ref, q_ref, k_ref, v_ref,
             qd_ref, kd_ref):
        f = f_ref[...]
        z = jnp.dot(f, w_ref[...], preferred_element_type=jnp.float32)
        z = z + b_ref[...][None, :]
        q = z[:, :DIM] * SCALE_Q
        k = z[:, DIM:2 * DIM]
        q_ref[...] = q
        k_ref[...] = k
        v_ref[...] = z[:, 2 * DIM:]
        qd_ref[...] = jnp.dot(q, tq_ref[...], preferred_element_type=jnp.float32)
        kd_ref[...] = jnp.dot(k, tk_ref[...], preferred_element_type=jnp.float32)

    full = lambda s: pl.BlockSpec(s, lambda i: (0, 0))
    row = lambda w: pl.BlockSpec((blk, w), lambda i: (i, 0))
    return pl.pallas_call(
        body,
        grid=(grid,),
        in_specs=[row(DIM), full((DIM, 3 * DIM)),
                  pl.BlockSpec((3 * DIM,), lambda i: (0,)),
                  full((DIM, 768)), full((DIM, 768))],
        out_specs=[row(DIM), row(DIM), row(DIM), row(768), row(768)],
        out_shape=[
            jax.ShapeDtypeStruct((N, DIM), jnp.float32),
            jax.ShapeDtypeStruct((N, DIM), jnp.float32),
            jax.ShapeDtypeStruct((N, DIM), jnp.float32),
            jax.ShapeDtypeStruct((N, 768), jnp.float32),
            jax.ShapeDtypeStruct((N, 768), jnp.float32),
        ],
    )(feats, qkv_wt, qkv_b, tq, tk)


def _tc_proj(x, proj_wt, proj_b):
    blk = 1000

    def body(x_ref, w_ref, b_ref, o_ref):
        o_ref[...] = (
            jnp.dot(x_ref[...], w_ref[...], preferred_element_type=jnp.float32)
            + b_ref[...][None, :])

    return pl.pallas_call(
        body,
        grid=(N // blk,),
        in_specs=[pl.BlockSpec((blk, DIM), lambda i: (i, 0)),
                  pl.BlockSpec((DIM, DIM), lambda i: (0, 0)),
                  pl.BlockSpec((DIM,), lambda i: (0,))],
        out_specs=pl.BlockSpec((blk, DIM), lambda i: (i, 0)),
        out_shape=jax.ShapeDtypeStruct((N, DIM), jnp.float32),
    )(x, proj_wt, proj_b)


def _sc_main(offs_pad, idx0_pad, idx1_pad, q2, k2, v2, xyz8, qd2, kd2):
    mesh = plsc.VectorSubcoreMesh(core_axis_name="c", subcore_axis_name="s",
                                  num_cores=2, num_subcores=16)
    f32, i32 = jnp.float32, jnp.int32

    @functools.partial(
        pl.kernel,
        out_type=jax.ShapeDtypeStruct((NW * QPW * DIM,), f32),
        mesh=mesh,
        compiler_params=pltpu.CompilerParams(
            needs_layout_passes=False, use_tc_tiling_on_sc=False),
        scratch_types=[
            pltpu.VMEM((352,), i32),
            pltpu.VMEM((LMAX,), i32),
            pltpu.VMEM((LMAX,), i32),
            [pltpu.VMEM((P, DIM), f32) for _ in range(2)],
            [pltpu.VMEM((P, DIM), f32) for _ in range(2)],
            pltpu.VMEM((P, DIM), f32),
            [pltpu.VMEM((P, 16), f32) for _ in range(2)],
            [pltpu.VMEM((P, 16), f32) for _ in range(2)],
            [pltpu.VMEM((P,), i32) for _ in range(3)],
            [pltpu.VMEM((P,), i32) for _ in range(3)],
            [pltpu.VMEM((P, 16), f32) for _ in range(3)],
            [pltpu.VMEM((P, 16), f32) for _ in range(3)],
            pltpu.VMEM((QPW * DIM + 128,), f32),
            pltpu.VMEM((QPW * 16 + 16,), f32),
            pltpu.SemaphoreType.DMA,
            pltpu.SemaphoreType.DMA,
            pltpu.SemaphoreType.DMA,
        ],
    )
    def body(offs_hbm, idx0_hbm, idx1_hbm, q2_hbm, k2_hbm, v2_hbm,
             xyz_hbm, qd_hbm, kd_hbm, out_hbm,
             offs_v, idx0_v, idx1_v, qb2, kb2, vb, xy02, xy12,
             qdi, kdi, qdb, kdb, xacc, sacc, sem0, sem1, sem2):
        LANE = jnp.arange(16, dtype=i32)
        cc = lax.axis_index("c")
        ss = lax.axis_index("s")
        wid = ss * 2 + cc
        qlo = wid * QPW
        qhi = jnp.minimum(qlo + QPW, N)
        base = (qlo // 8) * 8
        pltpu.sync_copy(offs_hbm.at[pl.ds(base, 352)], offs_v)
        plo = offs_v[pl.ds(qlo - base, 16)][0]
        phi = offs_v[pl.ds(qhi - base, 16)][0]
        pbase = (plo // 8) * 8
        off0 = plo - pbase
        jend = off0 + phi - plo
        pltpu.sync_copy(idx0_hbm.at[pl.ds(pbase, LMAX)], idx0_v)
        pltpu.sync_copy(idx1_hbm.at[pl.ds(pbase, LMAX)], idx1_v)

        zf = jnp.zeros((16,), f32)

        def zx(i, carry):
            xacc[pl.ds(i * 16, 16)] = zf
            return carry

        lax.fori_loop(0, (QPW * DIM + 128) // 16, zx, 0)

        def zs(i, carry):
            sacc[pl.ds(i * 16, 16)] = zf
            return carry

        lax.fori_loop(0, (QPW * 16 + 16) // 16, zs, 0)

        nchunks = (jend + P - 1) // P
        rowsems = [sem0, sem2]

        def row_cps(bi, ci, make):
            mk = pltpu.make_async_copy if make else pltpu.async_copy
            sl = pl.ds(ci * P, P)
            return [
                mk(q2_hbm.at[idx0_v.at[sl]], qb2[bi], rowsems[bi]),
                mk(k2_hbm.at[idx1_v.at[sl]], kb2[bi], rowsems[bi]),
                mk(xyz_hbm.at[idx0_v.at[sl]], xy02[bi], rowsems[bi]),
                mk(xyz_hbm.at[idx1_v.at[sl]], xy12[bi], rowsems[bi]),
            ]

        @pl.when(nchunks > 0)
        def _():
            row_cps(0, 0, False)

        @pl.when(nchunks > 1)
        def _():
            row_cps(1, 1, False)

        def compute(ci, bi):
            qb, kb, xy0, xy1 = qb2[bi], kb2[bi], xy02[bi], xy12[bi]
            vcp = pltpu.async_copy(
                v2_hbm.at[idx1_v.at[pl.ds(ci * P, P)]], vb, sem1)
            for cp in row_cps(bi, ci, True):
                cp.wait()
            for g in range(NG):
                j16 = g * 16 + LANE
                i0 = idx0_v[pl.ds(ci * P + g * 16, 16)]
                i1 = idx1_v[pl.ds(ci * P + g * 16, 16)]
                for c3 in range(3):
                    cvec = jnp.full((16,), c3, i32)
                    x0 = plsc.load_gather(xy0, [j16, cvec])
                    x1 = plsc.load_gather(xy1, [j16, cvec])
                    rel = x0 - x1
                    y = rel * 100000.0
                    r = (y + jnp.sign(y) * 0.5).astype(i32).astype(f32)
                    u = (r * 1e-5 + (2.0 * WINDOW - 0.0001)) * (1.0 / QUANT)
                    t = jnp.clip(u.astype(i32), 0, TBL - 1)
                    qdi[c3][pl.ds(g * 16, 16)] = i0 * 96 + c3 * TBL + t
                    kdi[c3][pl.ds(g * 16, 16)] = i1 * 96 + c3 * TBL + t
            cps = [pltpu.async_copy(qd_hbm.at[qdi[c3]], qdb[c3], sem1)
                   for c3 in range(3)]
            cps += [pltpu.async_copy(kd_hbm.at[kdi[c3]], kdb[c3], sem1)
                    for c3 in range(3)]
            for cp in cps:
                cp.wait()
            vcp.wait()
            for g in range(NG):
                j16 = g * 16 + LANE
                i0 = idx0_v[pl.ds(ci * P + g * 16, 16)]
                jj = ci * P + g * 16 + LANE
                ok = (jj >= off0) & (jj < jend)
                seg = jnp.where(ok, i0 - qlo, 0)
                accs = [zf] * NUM_HEADS
                for d in range(DIM):
                    dv = jnp.full((16,), d, i32)
                    qv = plsc.load_gather(qb, [j16, dv])
                    kv = plsc.load_gather(kb, [j16, dv])
                    accs[d // HEAD_DIM] = accs[d // HEAD_DIM] + qv * kv
                es = []
                for h in range(NUM_HEADS):
                    hv = jnp.full((16,), h, i32)
                    bias = zf
                    for c3 in range(3):
                        bias = bias + plsc.load_gather(qdb[c3], [j16, hv])
                        bias = bias + plsc.load_gather(kdb[c3], [j16, hv])
                    e = jnp.exp(accs[h] + bias)
                    es.append(jnp.where(ok, e, 0.0))
                onehots = [jnp.where(LANE == h, 1.0, 0.0).astype(f32)
                           for h in range(NUM_HEADS)]
                for j in range(16):
                    segj = seg[j]
                    rowb = segj * DIM
                    svec = zf
                    for h in range(NUM_HEADS):
                        svec = svec + es[h][j] * onehots[h]
                    plsc.addupdate(sacc.at[pl.ds(segj * 16, 16)], svec)
                    for h in range(NUM_HEADS):
                        vrow = vb[g * 16 + j, pl.ds(h * 16, 16)]
                        plsc.addupdate(xacc.at[pl.ds(rowb + h * 16, 16)],
                                       es[h][j] * vrow)

        def chunk2(c2, carry):
            for bi in range(2):
                ci = c2 * 2 + bi

                @pl.when(ci < nchunks)
                def _(ci=ci, bi=bi):
                    compute(ci, bi)

                    @pl.when(ci + 2 < nchunks)
                    def _():
                        row_cps(bi, ci + 2, False)
            return carry

        lax.fori_loop(0, (nchunks + 1) // 2, chunk2, 0)

        def fin(i, carry):
            s16 = sacc[pl.ds(i * 16, 16)]
            w16 = 1.0 / jnp.maximum(s16, 1e-30)
            for h in range(NUM_HEADS):
                sl = pl.ds(i * DIM + h * 16, 16)
                xacc[sl] = xacc[sl] * w16[h]
            return carry

        lax.fori_loop(0, QPW, fin, 0)
        pltpu.sync_copy(xacc.at[pl.ds(0, QPW * DIM)],
                        out_hbm.at[pl.ds(qlo * DIM, QPW * DIM)])

    return body(offs_pad, idx0_pad, idx1_pad, q2, k2, v2, xyz8, qd2, kd2)


def kernel(feats, xyz, index_0, index_1, index_0_offsets, n_max,
           qkv_w, qkv_b, proj_w, proj_b, rel_q_table, rel_k_table):
    f32, i32 = jnp.float32, jnp.int32
    eye = jnp.eye(NUM_HEADS, dtype=f32)
    tq = jnp.einsum("cthd,hg->hdctg",
                    jnp.transpose(rel_q_table, (3, 0, 1, 2)), eye)
    tq = tq.reshape(DIM, 3 * TBL * NUM_HEADS)
    tk = jnp.einsum("cthd,hg->hdctg",
                    jnp.transpose(rel_k_table, (3, 0, 1, 2)), eye)
    tk = tk.reshape(DIM, 3 * TBL * NUM_HEADS)

    q2, k2, v2, qd, kd = _tc_prep(feats, qkv_w.T, qkv_b, tq, tk)
    qd2 = jnp.pad(qd.reshape(N * 96, 8), ((0, 0), (0, 8)))
    kd2 = jnp.pad(kd.reshape(N * 96, 8), ((0, 0), (0, 8)))

    xyz8 = jnp.pad(xyz, ((0, 0), (0, 13)))
    offs_pad = jnp.pad(index_0_offsets.astype(i32), (0, 352))
    idx0_pad = jnp.pad(index_0.astype(i32), (0, LMAX + 8))
    idx1_pad = jnp.pad(index_1.astype(i32), (0, LMAX + 8))

    xraw = _sc_main(offs_pad, idx0_pad, idx1_pad, q2, k2, v2, xyz8, qd2, kd2)
    return _tc_proj(xraw.reshape(NPAD, DIM)[:N], proj_w.T, proj_b)


# speedup vs baseline: 18.8023x; 1.0607x over previous
import functools

import jax
import jax.numpy as jnp
from jax import lax
from jax.experimental import pallas as pl
from jax.experimental.pallas import tpu as pltpu, tpu_sc as plsc

N = 10000
DIM = 128
NUM_HEADS = 8
HEAD_DIM = 16
M = 320000
SCALE_Q = float(HEAD_DIM) ** -0.5
QUANT = 0.075
WINDOW = 0.6
NQT = 16
TBL = 2 * NQT
NW = 32
QPW = 313
LMAX = 10880
P = 32
NG = P // 16
NPAD = NW * QPW


def _tc_prep(feats, qkv_wt, qkv_b, tq, tk):
    blk = 1000
    grid = N // blk

    def body(f_ref, w_ref, b_ref, tq_ref, tk_ref, q_ref, k_ref, v_ref,
             qd_ref, kd_ref):
        f = f_ref[...]
        z = jnp.dot(f, w_ref[...], preferred_element_type=jnp.float32)
        z = z + b_ref[...][None, :]
        q = z[:, :DIM] * SCALE_Q
        k = z[:, DIM:2 * DIM]
        q_ref[...] = q
        k_ref[...] = k
        v_ref[...] = z[:, 2 * DIM:]
        qd_ref[...] = jnp.dot(q, tq_ref[...], preferred_element_type=jnp.float32)
        kd_ref[...] = jnp.dot(k, tk_ref[...], preferred_element_type=jnp.float32)

    full = lambda s: pl.BlockSpec(s, lambda i: (0, 0))
    row = lambda w: pl.BlockSpec((blk, w), lambda i: (i, 0))
    return pl.pallas_call(
        body,
        grid=(grid,),
        in_specs=[row(DIM), full((DIM, 3 * DIM)),
                  pl.BlockSpec((3 * DIM,), lambda i: (0,)),
                  full((DIM, 768)), full((DIM, 768))],
        out_specs=[row(DIM), row(DIM), row(DIM), row(768), row(768)],
        out_shape=[
            jax.ShapeDtypeStruct((N, DIM), jnp.float32),
            jax.ShapeDtypeStruct((N, DIM), jnp.float32),
            jax.ShapeDtypeStruct((N, DIM), jnp.float32),
            jax.ShapeDtypeStruct((N, 768), jnp.float32),
            jax.ShapeDtypeStruct((N, 768), jnp.float32),
        ],
    )(feats, qkv_wt, qkv_b, tq, tk)


def _tc_proj(x, proj_wt, proj_b):
    blk = 1000

    def body(x_ref, w_ref, b_ref, o_ref):
        o_ref[...] = (
            jnp.dot(x_ref[...], w_ref[...], preferred_element_type=jnp.float32)
            + b_ref[...][None, :])

    return pl.pallas_call(
        body,
        grid=(N // blk,),
        in_specs=[pl.BlockSpec((blk, DIM), lambda i: (i, 0)),
                  pl.BlockSpec((DIM, DIM), lambda i: (0, 0)),
                  pl.BlockSpec((DIM,), lambda i: (0,))],
        out_specs=pl.BlockSpec((blk, DIM), lambda i: (i, 0)),
        out_shape=jax.ShapeDtypeStruct((N, DIM), jnp.float32),
    )(x, proj_wt, proj_b)


def _sc_main(offs_pad, idx0_pad, idx1_pad, q2, k2, v2, xyz8, qd2, kd2):
    mesh = plsc.VectorSubcoreMesh(core_axis_name="c", subcore_axis_name="s",
                                  num_cores=2, num_subcores=16)
    f32, i32 = jnp.float32, jnp.int32

    @functools.partial(
        pl.kernel,
        out_type=jax.ShapeDtypeStruct((NW * QPW * DIM,), f32),
        mesh=mesh,
        compiler_params=pltpu.CompilerParams(
            needs_layout_passes=False, use_tc_tiling_on_sc=False),
        scratch_types=[
            pltpu.VMEM((352,), i32),
            pltpu.VMEM((LMAX,), i32),
            pltpu.VMEM((LMAX,), i32),
            [pltpu.VMEM((P, DIM), f32) for _ in range(2)],
            [pltpu.VMEM((P, DIM), f32) for _ in range(2)],
            pltpu.VMEM((P, DIM), f32),
            [pltpu.VMEM((P, 16), f32) for _ in range(2)],
            [pltpu.VMEM((P, 16), f32) for _ in range(2)],
            [pltpu.VMEM((P,), i32) for _ in range(3)],
            [pltpu.VMEM((P,), i32) for _ in range(3)],
            [pltpu.VMEM((P, 16), f32) for _ in range(3)],
            [pltpu.VMEM((P, 16), f32) for _ in range(3)],
            pltpu.VMEM((QPW * DIM + 128,), f32),
            pltpu.VMEM((QPW * 16 + 16,), f32),
            pltpu.SemaphoreType.DMA,
            pltpu.SemaphoreType.DMA,
            pltpu.SemaphoreType.DMA,
        ],
    )
    def body(offs_hbm, idx0_hbm, idx1_hbm, q2_hbm, k2_hbm, v2_hbm,
             xyz_hbm, qd_hbm, kd_hbm, out_hbm,
             offs_v, idx0_v, idx1_v, qb2, kb2, vb, xy02, xy12,
             qdi, kdi, qdb, kdb, xacc, sacc, sem0, sem1, sem2):
        LANE = jnp.arange(16, dtype=i32)
        cc = lax.axis_index("c")
        ss = lax.axis_index("s")
        wid = ss * 2 + cc
        qlo = wid * QPW
        qhi = jnp.minimum(qlo + QPW, N)
        base = (qlo // 8) * 8
        pltpu.sync_copy(offs_hbm.at[pl.ds(base, 352)], offs_v)
        plo = offs_v[pl.ds(qlo - base, 16)][0]
        phi = offs_v[pl.ds(qhi - base, 16)][0]
        pbase = (plo // 8) * 8
        off0 = plo - pbase
        jend = off0 + phi - plo
        pltpu.sync_copy(idx0_hbm.at[pl.ds(pbase, LMAX)], idx0_v)
        pltpu.sync_copy(idx1_hbm.at[pl.ds(pbase, LMAX)], idx1_v)

        zf = jnp.zeros((16,), f32)

        def zx(i, carry):
            xacc[pl.ds(i * 16, 16)] = zf
            return carry

        lax.fori_loop(0, (QPW * DIM + 128) // 16, zx, 0)

        def zs(i, carry):
            sacc[pl.ds(i * 16, 16)] = zf
            return carry

        lax.fori_loop(0, (QPW * 16 + 16) // 16, zs, 0)

        nchunks = (jend + P - 1) // P
        rowsems = [sem0, sem2]

        def row_cps(bi, ci, make):
            mk = pltpu.make_async_copy if make else pltpu.async_copy
            sl = pl.ds(ci * P, P)
            return [
                mk(q2_hbm.at[idx0_v.at[sl]], qb2[bi], rowsems[bi]),
                mk(k2_hbm.at[idx1_v.at[sl]], kb2[bi], rowsems[bi]),
                mk(xyz_hbm.at[idx0_v.at[sl]], xy02[bi], rowsems[bi]),
                mk(xyz_hbm.at[idx1_v.at[sl]], xy12[bi], rowsems[bi]),
            ]

        @pl.when(nchunks > 0)
        def _():
            row_cps(0, 0, False)

        @pl.when(nchunks > 1)
        def _():
            row_cps(1, 1, False)

        def compute(ci, bi):
            qb, kb, xy0, xy1 = qb2[bi], kb2[bi], xy02[bi], xy12[bi]
            vcp = pltpu.async_copy(
                v2_hbm.at[idx1_v.at[pl.ds(ci * P, P)]], vb, sem1)
            for cp in row_cps(bi, ci, True):
                cp.wait()
            for g in range(NG):
                j16 = g * 16 + LANE
                i0 = idx0_v[pl.ds(ci * P + g * 16, 16)]
                i1 = idx1_v[pl.ds(ci * P + g * 16, 16)]
                for c3 in range(3):
                    cvec = jnp.full((16,), c3, i32)
                    x0 = plsc.load_gather(xy0, [j16, cvec])
                    x1 = plsc.load_gather(xy1, [j16, cvec])
                    rel = x0 - x1
                    y = rel * 100000.0
                    r = (y + jnp.sign(y) * 0.5).astype(i32).astype(f32)
                    u = (r * 1e-5 + (2.0 * WINDOW - 0.0001)) * (1.0 / QUANT)
                    t = jnp.clip(u.astype(i32), 0, TBL - 1)
                    qdi[c3][pl.ds(g * 16, 16)] = i0 * 96 + c3 * TBL + t
                    kdi[c3][pl.ds(g * 16, 16)] = i1 * 96 + c3 * TBL + t
            cps = [pltpu.async_copy(qd_hbm.at[qdi[c3]], qdb[c3], sem1)
                   for c3 in range(3)]
            cps += [pltpu.async_copy(kd_hbm.at[kdi[c3]], kdb[c3], sem1)
                    for c3 in range(3)]
            for cp in cps:
                cp.wait()
            vcp.wait()
            onehots = [jnp.where(LANE == h, 1.0, 0.0).astype(f32)
                       for h in range(NUM_HEADS)]
            hmask = jnp.where(LANE < 8, 1.0, 0.0).astype(f32)
            for g in range(NG):
                i0 = idx0_v[pl.ds(ci * P + g * 16, 16)]
                jj = ci * P + g * 16 + LANE
                ok = (jj >= off0) & (jj < jend)
                seg = jnp.where(ok, i0 - qlo, 0)
                okf = jnp.where(ok, 1.0, 0.0).astype(f32)
                for j in range(16):
                    jr = g * 16 + j
                    lg = zf
                    for h in range(NUM_HEADS):
                        qv = qb[jr, pl.ds(h * 16, 16)]
                        kv = kb[jr, pl.ds(h * 16, 16)]
                        lg = lg + jnp.sum(qv * kv) * onehots[h]
                    bias = (qdb[0][jr, :] + qdb[1][jr, :] + qdb[2][jr, :]
                            + kdb[0][jr, :] + kdb[1][jr, :] + kdb[2][jr, :])
                    e16 = jnp.exp(lg + bias) * (hmask * okf[j])
                    segj = seg[j]
                    rowb = segj * DIM
                    plsc.addupdate(sacc.at[pl.ds(segj * 16, 16)], e16)
                    for h in range(NUM_HEADS):
                        vrow = vb[jr, pl.ds(h * 16, 16)]
                        plsc.addupdate(xacc.at[pl.ds(rowb + h * 16, 16)],
                                       e16[h] * vrow)

        def chunk2(c2, carry):
            for bi in range(2):
                ci = c2 * 2 + bi

                @pl.when(ci < nchunks)
                def _(ci=ci, bi=bi):
                    compute(ci, bi)

                    @pl.when(ci + 2 < nchunks)
                    def _():
                        row_cps(bi, ci + 2, False)
            return carry

        lax.fori_loop(0, (nchunks + 1) // 2, chunk2, 0)

        def fin(i, carry):
            s16 = sacc[pl.ds(i * 16, 16)]
            w16 = 1.0 / jnp.maximum(s16, 1e-30)
            for h in range(NUM_HEADS):
                sl = pl.ds(i * DIM + h * 16, 16)
                xacc[sl] = xacc[sl] * w16[h]
            return carry

        lax.fori_loop(0, QPW, fin, 0)
        pltpu.sync_copy(xacc.at[pl.ds(0, QPW * DIM)],
                        out_hbm.at[pl.ds(qlo * DIM, QPW * DIM)])

    return body(offs_pad, idx0_pad, idx1_pad, q2, k2, v2, xyz8, qd2, kd2)


def kernel(feats, xyz, index_0, index_1, index_0_offsets, n_max,
           qkv_w, qkv_b, proj_w, proj_b, rel_q_table, rel_k_table):
    f32, i32 = jnp.float32, jnp.int32
    eye = jnp.eye(NUM_HEADS, dtype=f32)
    tq = jnp.einsum("cthd,hg->hdctg",
                    jnp.transpose(rel_q_table, (3, 0, 1, 2)), eye)
    tq = tq.reshape(DIM, 3 * TBL * NUM_HEADS)
    tk = jnp.einsum("cthd,hg->hdctg",
                    jnp.transpose(rel_k_table, (3, 0, 1, 2)), eye)
    tk = tk.reshape(DIM, 3 * TBL * NUM_HEADS)

    q2, k2, v2, qd, kd = _tc_prep(feats, qkv_w.T, qkv_b, tq, tk)
    qd2 = jnp.pad(qd.reshape(N * 96, 8), ((0, 0), (0, 8)))
    kd2 = jnp.pad(kd.reshape(N * 96, 8), ((0, 0), (0, 8)))

    xyz8 = jnp.pad(xyz, ((0, 0), (0, 13)))
    offs_pad = jnp.pad(index_0_offsets.astype(i32), (0, 352))
    idx0_pad = jnp.pad(index_0.astype(i32), (0, LMAX + 8))
    idx1_pad = jnp.pad(index_1.astype(i32), (0, LMAX + 8))

    xraw = _sc_main(offs_pad, idx0_pad, idx1_pad, q2, k2, v2, xyz8, qd2, kd2)
    return _tc_proj(xraw.reshape(NPAD, DIM)[:N], proj_w.T, proj_b)
